# baseline (speedup 1.0000x reference)
import functools

import jax
import jax.numpy as jnp
from jax import lax
from jax.experimental import pallas as pl
from jax.experimental.pallas import tpu as pltpu

DIM = 1024
E = 8
K = 2
H = 16
EH = E * H

TM1 = 2048
TM2 = 1024


def _pass1_body(x_ref, g1_ref, stats_ref):
    h = lax.dot_general(x_ref[...], g1_ref[...], (((1,), (1,)), ((), ())),
                        preferred_element_type=jnp.float32)
    s1 = jnp.sum(h, axis=0, keepdims=True)
    s2 = jnp.sum(h * h, axis=0, keepdims=True)
    both = jnp.concatenate([s1, s2], axis=0)

    @pl.when(pl.program_id(0) == 0)
    def _init():
        stats_ref[...] = jnp.zeros_like(stats_ref)

    stats_ref[0:2, :] += both


def _pass2_body(x_ref, stats_ref, g1_ref, gamma_ref, beta_ref, g2_ref, g2p_ref,
                w1c_ref, b1p_ref, w2_ref, b2_ref, out_ref, lb_ref, acc_ref,
                *, total_tokens):
    i = pl.program_id(0)
    inv_t = 1.0 / total_tokens
    mean = stats_ref[0:1, :] * inv_t
    msq = stats_ref[1:2, :] * inv_t
    var = msq - mean * mean
    inv = 1.0 / jnp.sqrt(var + 1e-5)
    x = x_ref[...]
    h = lax.dot_general(x, g1_ref[...], (((1,), (1,)), ((), ())),
                        preferred_element_type=jnp.float32)
    hn = (h - mean) * (inv * gamma_ref[...]) + beta_ref[...]
    hn = jnp.maximum(hn, 0.0)
    logits = lax.dot_general(g2_ref[...], hn, (((1,), (1,)), ((), ())),
                             preferred_element_type=jnp.float32) + g2p_ref[:, 0:1]

    ie = lax.broadcasted_iota(jnp.int32, (E, TM2), 0)
    m1 = jnp.max(logits, axis=0, keepdims=True)
    i1 = jnp.min(jnp.where(logits == m1, ie, E), axis=0, keepdims=True)
    neg = jnp.where(ie == i1, -jnp.inf, logits)
    m2 = jnp.max(neg, axis=0, keepdims=True)
    i2 = jnp.min(jnp.where(neg == m2, ie, E), axis=0, keepdims=True)
    e2 = jnp.exp(m2 - m1)
    w1v = 1.0 / (1.0 + e2)
    w2v = e2 * w1v
    zero = jnp.zeros_like(logits)
    weight = jnp.where(ie == i1, w1v, zero) + jnp.where(ie == i2, w2v, zero)

    ieh = lax.broadcasted_iota(jnp.int32, (EH, TM2), 0) // H
    zeroh = jnp.zeros((EH, TM2), jnp.float32)
    wrep = jnp.where(ieh == i1, w1v, zeroh) + jnp.where(ieh == i2, w2v, zeroh)

    x16 = x.astype(jnp.bfloat16)
    het = lax.dot_general(w1c_ref[...], x16, (((1,), (1,)), ((), ())),
                          preferred_element_type=jnp.float32) + b1p_ref[:, 0:1]
    het = jnp.maximum(het, 0.0)
    hw = (het * wrep).astype(jnp.bfloat16)
    out = lax.dot_general(w2_ref[...], hw, (((0,), (0,)), ((), ())),
                          preferred_element_type=jnp.float32)
    out = out + lax.dot_general(b2_ref[...], weight, (((0,), (0,)), ((), ())),
                                preferred_element_type=jnp.float32)
    out_ref[0] = out

    part = jnp.sum(weight, axis=1, keepdims=True)

    @pl.when(i == 0)
    def _init():
        acc_ref[...] = jnp.zeros_like(acc_ref)

    acc_ref[0:E, 0:1] += part

    @pl.when(i == pl.num_programs(0) - 1)
    def _fin():
        u = acc_ref[0:E, 0:1] * inv_t
        lb_ref[...] = jnp.sum(u * u, axis=0, keepdims=True) * float(E)


def kernel(inputs, W1, b1, W2, b2, G1, g1b, bn_gamma, bn_beta, G2, g2b):
    Bv, Nv, C = inputs.shape
    T = Bv * Nv
    flat = inputs.reshape(T, C)
    w1c = W1.transpose(0, 2, 1).reshape(EH, C).astype(jnp.bfloat16)
    b1p = jnp.zeros((EH, 128), jnp.float32).at[:, 0].set(b1.reshape(EH))
    w2r = W2.reshape(EH, C).astype(jnp.bfloat16)
    gammar = bn_gamma.reshape(1, C)
    betar = bn_beta.reshape(1, C)
    g2p = jnp.zeros((E, 128), jnp.float32).at[:, 0].set(g2b)

    n1 = T // TM1
    stats = pl.pallas_call(
        _pass1_body,
        grid=(n1,),
        in_specs=[
            pl.BlockSpec((TM1, C), lambda i: (i, 0)),
            pl.BlockSpec((C, C), lambda i: (0, 0)),
        ],
        out_specs=pl.BlockSpec((8, C), lambda i: (0, 0)),
        out_shape=jax.ShapeDtypeStruct((8, C), jnp.float32),
    )(flat, G1)

    n2 = T // TM2
    tiles_per_batch = Nv // TM2
    out_t, lb = pl.pallas_call(
        functools.partial(_pass2_body, total_tokens=float(T)),
        grid=(n2,),
        in_specs=[
            pl.BlockSpec((TM2, C), lambda i: (i, 0)),
            pl.BlockSpec((8, C), lambda i: (0, 0)),
            pl.BlockSpec((C, C), lambda i: (0, 0)),
            pl.BlockSpec((1, C), lambda i: (0, 0)),
            pl.BlockSpec((1, C), lambda i: (0, 0)),
            pl.BlockSpec((E, C), lambda i: (0, 0)),
            pl.BlockSpec((E, 128), lambda i: (0, 0)),
            pl.BlockSpec((EH, C), lambda i: (0, 0)),
            pl.BlockSpec((EH, 128), lambda i: (0, 0)),
            pl.BlockSpec((EH, C), lambda i: (0, 0)),
            pl.BlockSpec((E, C), lambda i: (0, 0)),
        ],
        out_specs=[
            pl.BlockSpec((1, C, TM2),
                         lambda i, tpb=tiles_per_batch: (i // tpb, 0, i % tpb)),
            pl.BlockSpec((1, 1), lambda i: (0, 0)),
        ],
        out_shape=[
            jax.ShapeDtypeStruct((Bv, C, Nv), jnp.float32),
            jax.ShapeDtypeStruct((1, 1), jnp.float32),
        ],
        scratch_shapes=[pltpu.VMEM((8, 128), jnp.float32)],
    )(flat, stats, G1, gammar, betar, G2, g2p, w1c, b1p, w2r, b2)

    return (out_t, lb.reshape(()))


# speedup vs baseline: 1.0986x; 1.0986x over previous
import functools

import jax
import jax.numpy as jnp
from jax import lax
from jax.experimental import pallas as pl
from jax.experimental.pallas import tpu as pltpu

DIM = 1024
E = 8
K = 2
H = 16
EH = E * H

TM1 = 2048
TM2 = 1024


def _pass1_body(x_ref, g1_ref, w1c_ref, b1p_ref, h_ref, het_ref, stats_ref):
    x = x_ref[...]
    h = lax.dot_general(x, g1_ref[...], (((1,), (1,)), ((), ())),
                        preferred_element_type=jnp.float32)
    h_ref[...] = h
    x16 = x.astype(jnp.bfloat16)
    het = lax.dot_general(w1c_ref[...], x16, (((1,), (1,)), ((), ())),
                          preferred_element_type=jnp.float32) + b1p_ref[:, 0:1]
    het_ref[...] = jnp.maximum(het, 0.0).astype(jnp.bfloat16)
    s1 = jnp.sum(h, axis=0, keepdims=True)
    s2 = jnp.sum(h * h, axis=0, keepdims=True)
    both = jnp.concatenate([s1, s2], axis=0)

    @pl.when(pl.program_id(0) == 0)
    def _init():
        stats_ref[...] = jnp.zeros_like(stats_ref)

    stats_ref[0:2, :] += both


def _pass2_body(h_ref, het_ref, stats_ref, gamma_ref, beta_ref, g2_ref, g2p_ref,
                w2_ref, b2_ref, out_ref, lb_ref, acc_ref, *, total_tokens):
    i = pl.program_id(0)
    inv_t = 1.0 / total_tokens
    mean = stats_ref[0:1, :] * inv_t
    msq = stats_ref[1:2, :] * inv_t
    var = msq - mean * mean
    inv = 1.0 / jnp.sqrt(var + 1e-5)
    hn = (h_ref[...] - mean) * (inv * gamma_ref[...]) + beta_ref[...]
    hn = jnp.maximum(hn, 0.0)
    logits = lax.dot_general(g2_ref[...], hn, (((1,), (1,)), ((), ())),
                             preferred_element_type=jnp.float32) + g2p_ref[:, 0:1]

    ie = lax.broadcasted_iota(jnp.int32, (E, TM2), 0)
    m1 = jnp.max(logits, axis=0, keepdims=True)
    i1 = jnp.min(jnp.where(logits == m1, ie, E), axis=0, keepdims=True)
    neg = jnp.where(ie == i1, -jnp.inf, logits)
    m2 = jnp.max(neg, axis=0, keepdims=True)
    i2 = jnp.min(jnp.where(neg == m2, ie, E), axis=0, keepdims=True)
    e2 = jnp.exp(m2 - m1)
    w1v = 1.0 / (1.0 + e2)
    w2v = e2 * w1v
    zero = jnp.zeros_like(logits)
    weight = jnp.where(ie == i1, w1v, zero) + jnp.where(ie == i2, w2v, zero)

    ieh = lax.broadcasted_iota(jnp.int32, (EH, TM2), 0) // H
    zeroh = jnp.zeros((EH, TM2), jnp.float32)
    wrep = jnp.where(ieh == i1, w1v, zeroh) + jnp.where(ieh == i2, w2v, zeroh)

    hw = (het_ref[...].astype(jnp.float32) * wrep).astype(jnp.bfloat16)
    out = lax.dot_general(w2_ref[...], hw, (((0,), (0,)), ((), ())),
                          preferred_element_type=jnp.float32)
    out = out + lax.dot_general(b2_ref[...], weight, (((0,), (0,)), ((), ())),
                                preferred_element_type=jnp.float32)
    out_ref[0] = out

    part = jnp.sum(weight, axis=1, keepdims=True)

    @pl.when(i == 0)
    def _init():
        acc_ref[...] = jnp.zeros_like(acc_ref)

    acc_ref[0:E, 0:1] += part

    @pl.when(i == pl.num_programs(0) - 1)
    def _fin():
        u = acc_ref[0:E, 0:1] * inv_t
        lb_ref[...] = jnp.sum(u * u, axis=0, keepdims=True) * float(E)


def kernel(inputs, W1, b1, W2, b2, G1, g1b, bn_gamma, bn_beta, G2, g2b):
    Bv, Nv, C = inputs.shape
    T = Bv * Nv
    flat = inputs.reshape(T, C)
    w1c = W1.transpose(0, 2, 1).reshape(EH, C).astype(jnp.bfloat16)
    b1p = jnp.zeros((EH, 128), jnp.float32).at[:, 0].set(b1.reshape(EH))
    w2r = W2.reshape(EH, C).astype(jnp.bfloat16)
    gammar = bn_gamma.reshape(1, C)
    betar = bn_beta.reshape(1, C)
    g2p = jnp.zeros((E, 128), jnp.float32).at[:, 0].set(g2b)

    n1 = T // TM1
    h, het, stats = pl.pallas_call(
        _pass1_body,
        grid=(n1,),
        in_specs=[
            pl.BlockSpec((TM1, C), lambda i: (i, 0)),
            pl.BlockSpec((C, C), lambda i: (0, 0)),
            pl.BlockSpec((EH, C), lambda i: (0, 0)),
            pl.BlockSpec((EH, 128), lambda i: (0, 0)),
        ],
        out_specs=[
            pl.BlockSpec((TM1, C), lambda i: (i, 0)),
            pl.BlockSpec((EH, TM1), lambda i: (0, i)),
            pl.BlockSpec((8, C), lambda i: (0, 0)),
        ],
        out_shape=[
            jax.ShapeDtypeStruct((T, C), jnp.float32),
            jax.ShapeDtypeStruct((EH, T), jnp.bfloat16),
            jax.ShapeDtypeStruct((8, C), jnp.float32),
        ],
    )(flat, G1, w1c, b1p)

    n2 = T // TM2
    tiles_per_batch = Nv // TM2
    out_t, lb = pl.pallas_call(
        functools.partial(_pass2_body, total_tokens=float(T)),
        grid=(n2,),
        in_specs=[
            pl.BlockSpec((TM2, C), lambda i: (i, 0)),
            pl.BlockSpec((EH, TM2), lambda i: (0, i)),
            pl.BlockSpec((8, C), lambda i: (0, 0)),
            pl.BlockSpec((1, C), lambda i: (0, 0)),
            pl.BlockSpec((1, C), lambda i: (0, 0)),
            pl.BlockSpec((E, C), lambda i: (0, 0)),
            pl.BlockSpec((E, 128), lambda i: (0, 0)),
            pl.BlockSpec((EH, C), lambda i: (0, 0)),
            pl.BlockSpec((E, C), lambda i: (0, 0)),
        ],
        out_specs=[
            pl.BlockSpec((1, C, TM2),
                         lambda i, tpb=tiles_per_batch: (i // tpb, 0, i % tpb)),
            pl.BlockSpec((1, 1), lambda i: (0, 0)),
        ],
        out_shape=[
            jax.ShapeDtypeStruct((Bv, C, Nv), jnp.float32),
            jax.ShapeDtypeStruct((1, 1), jnp.float32),
        ],
        scratch_shapes=[pltpu.VMEM((8, 128), jnp.float32)],
    )(h, het, stats, gammar, betar, G2, g2p, w2r, b2)

    return (out_t, lb.reshape(()))


# speedup vs baseline: 1.1627x; 1.0584x over previous
import functools

import jax
import jax.numpy as jnp
from jax import lax
from jax.experimental import pallas as pl
from jax.experimental.pallas import tpu as pltpu

DIM = 1024
E = 8
K = 2
H = 16
EH = E * H

TM1 = 1024
TM2 = 1024


def _pass1_body(x_ref, g1_ref, w1c_ref, b1p_ref, h_ref, het_ref, stats_ref):
    x = x_ref[...]
    h = lax.dot_general(x, g1_ref[...], (((1,), (1,)), ((), ())),
                        preferred_element_type=jnp.float32)
    h_ref[...] = h
    het = lax.dot_general(w1c_ref[...], x, (((1,), (1,)), ((), ())),
                          preferred_element_type=jnp.float32) + b1p_ref[:, 0:1]
    het_ref[...] = jnp.maximum(het, 0.0)
    s1 = jnp.sum(h, axis=0, keepdims=True)
    s2 = jnp.sum(h * h, axis=0, keepdims=True)
    both = jnp.concatenate([s1, s2], axis=0)

    @pl.when(pl.program_id(0) == 0)
    def _init():
        stats_ref[...] = jnp.zeros_like(stats_ref)

    stats_ref[0:2, :] += both


def _pass2_body(h_ref, het_ref, stats_ref, gamma_ref, beta_ref, g2_ref, g2p_ref,
                w2_ref, b2_ref, out_ref, lb_ref, acc_ref, *, total_tokens):
    i = pl.program_id(0)
    inv_t = 1.0 / total_tokens
    mean = stats_ref[0:1, :] * inv_t
    msq = stats_ref[1:2, :] * inv_t
    var = msq - mean * mean
    inv = 1.0 / jnp.sqrt(var + 1e-5)
    hn = (h_ref[...] - mean) * (inv * gamma_ref[...]) + beta_ref[...]
    hn = jnp.maximum(hn, 0.0)
    logits = lax.dot_general(g2_ref[...], hn, (((1,), (1,)), ((), ())),
                             preferred_element_type=jnp.float32) + g2p_ref[:, 0:1]

    ie = lax.broadcasted_iota(jnp.int32, (E, TM2), 0)
    m1 = jnp.max(logits, axis=0, keepdims=True)
    i1 = jnp.min(jnp.where(logits == m1, ie, E), axis=0, keepdims=True)
    neg = jnp.where(ie == i1, -jnp.inf, logits)
    m2 = jnp.max(neg, axis=0, keepdims=True)
    i2 = jnp.min(jnp.where(neg == m2, ie, E), axis=0, keepdims=True)
    e2 = jnp.exp(m2 - m1)
    w1v = 1.0 / (1.0 + e2)
    w2v = e2 * w1v
    zero = jnp.zeros_like(logits)
    weight = jnp.where(ie == i1, w1v, zero) + jnp.where(ie == i2, w2v, zero)

    ieh = lax.broadcasted_iota(jnp.int32, (EH, TM2), 0) // H
    zeroh = jnp.zeros((EH, TM2), jnp.float32)
    wrep = jnp.where(ieh == i1, w1v, zeroh) + jnp.where(ieh == i2, w2v, zeroh)

    hw = het_ref[...] * wrep
    out = lax.dot_general(w2_ref[...], hw, (((0,), (0,)), ((), ())),
                          preferred_element_type=jnp.float32)
    out = out + lax.dot_general(b2_ref[...], weight, (((0,), (0,)), ((), ())),
                                preferred_element_type=jnp.float32)
    out_ref[0] = out

    part = jnp.sum(weight, axis=1, keepdims=True)

    @pl.when(i == 0)
    def _init():
        acc_ref[...] = jnp.zeros_like(acc_ref)

    acc_ref[0:E, 0:1] += part

    @pl.when(i == pl.num_programs(0) - 1)
    def _fin():
        u = acc_ref[0:E, 0:1] * inv_t
        lb_ref[...] = jnp.sum(u * u, axis=0, keepdims=True) * float(E)


def kernel(inputs, W1, b1, W2, b2, G1, g1b, bn_gamma, bn_beta, G2, g2b):
    Bv, Nv, C = inputs.shape
    T = Bv * Nv
    flat = inputs.reshape(T, C)
    w1c = W1.transpose(0, 2, 1).reshape(EH, C)
    b1p = jnp.zeros((EH, 128), jnp.float32).at[:, 0].set(b1.reshape(EH))
    w2r = W2.reshape(EH, C)
    gammar = bn_gamma.reshape(1, C)
    betar = bn_beta.reshape(1, C)
    g2p = jnp.zeros((E, 128), jnp.float32).at[:, 0].set(g2b)

    n1 = T // TM1
    h, het, stats = pl.pallas_call(
        _pass1_body,
        grid=(n1,),
        in_specs=[
            pl.BlockSpec((TM1, C), lambda i: (i, 0)),
            pl.BlockSpec((C, C), lambda i: (0, 0)),
            pl.BlockSpec((EH, C), lambda i: (0, 0)),
            pl.BlockSpec((EH, 128), lambda i: (0, 0)),
        ],
        out_specs=[
            pl.BlockSpec((TM1, C), lambda i: (i, 0)),
            pl.BlockSpec((EH, TM1), lambda i: (0, i)),
            pl.BlockSpec((8, C), lambda i: (0, 0)),
        ],
        out_shape=[
            jax.ShapeDtypeStruct((T, C), jnp.float32),
            jax.ShapeDtypeStruct((EH, T), jnp.float32),
            jax.ShapeDtypeStruct((8, C), jnp.float32),
        ],
    )(flat, G1, w1c, b1p)

    n2 = T // TM2
    tiles_per_batch = Nv // TM2
    out_t, lb = pl.pallas_call(
        functools.partial(_pass2_body, total_tokens=float(T)),
        grid=(n2,),
        in_specs=[
            pl.BlockSpec((TM2, C), lambda i: (i, 0)),
            pl.BlockSpec((EH, TM2), lambda i: (0, i)),
            pl.BlockSpec((8, C), lambda i: (0, 0)),
            pl.BlockSpec((1, C), lambda i: (0, 0)),
            pl.BlockSpec((1, C), lambda i: (0, 0)),
            pl.BlockSpec((E, C), lambda i: (0, 0)),
            pl.BlockSpec((E, 128), lambda i: (0, 0)),
            pl.BlockSpec((EH, C), lambda i: (0, 0)),
            pl.BlockSpec((E, C), lambda i: (0, 0)),
        ],
        out_specs=[
            pl.BlockSpec((1, C, TM2),
                         lambda i, tpb=tiles_per_batch: (i // tpb, 0, i % tpb)),
            pl.BlockSpec((1, 1), lambda i: (0, 0)),
        ],
        out_shape=[
            jax.ShapeDtypeStruct((Bv, C, Nv), jnp.float32),
            jax.ShapeDtypeStruct((1, 1), jnp.float32),
        ],
        scratch_shapes=[pltpu.VMEM((8, 128), jnp.float32)],
    )(h, het, stats, gammar, betar, G2, g2p, w2r, b2)

    return (out_t, lb.reshape(()))
